# baseline (speedup 1.0000x reference)
import functools

import jax
import jax.numpy as jnp
from jax import lax
from jax.experimental import pallas as pl
from jax.experimental.pallas import tpu as pltpu
from jax.experimental.pallas import tpu_sc as plsc

N_ENT = 10000
E = 320000
D = 128
NUM_REL = 101
B = 1024

NC = 2
NS = 16
L = 16
NW = NC * NS

EROW = 128
EPT = 10240
E_PAD = NW * EPT
PAD_E = E_PAD - E
RPT = EPT // EROW

N_PAD = 10240
ZROW = 128
NZ = N_PAD // ZROW // NS

_mesh = plsc.VectorSubcoreMesh(
    core_axis_name="c", subcore_axis_name="s", num_cores=NC, num_subcores=NS)


def _zero_rows(buf, nrows, ncols):
    def body(i, _):
        for j in range(ncols // L):
            buf[i, pl.ds(j * L, L)] = jnp.zeros((L,), jnp.float32)
        return 0
    lax.fori_loop(0, nrows, body, 0)


def _make_sc_pass(with_rel):
    nf = 3 if with_rel else 2

    scratch = [pltpu.VMEM((nf, EROW), jnp.int32) for _ in range(2)]
    scratch += [pltpu.VMEM((EROW, D), jnp.float32)]
    if with_rel:
        scratch += [pltpu.VMEM((EROW, D), jnp.float32)]
    scratch += [pltpu.VMEM_SHARED((N_PAD, D), jnp.float32)]
    if with_rel:
        scratch += [pltpu.VMEM_SHARED((NUM_REL, D), jnp.float32)]
    scratch += [pltpu.SemaphoreType.DMA for _ in range(4 if with_rel else 3)]

    def body(*refs):
        if with_rel:
            (tab_hbm, rel_hbm, pidx_hbm, out_hbm, pi0, pi1,
             srows, rrows, agg, rel_sh, se, sr, si0, si1) = refs
        else:
            (tab_hbm, pidx_hbm, out_hbm, pi0, pi1,
             srows, agg, se, si0, si1) = refs
        pidx = (pi0, pi1)
        sidx = (si0, si1)

        c = lax.axis_index("c")
        s = lax.axis_index("s")
        wid = s * NC + c
        base = wid * RPT

        _zero_rows(srows, EROW, D)

        def zc(k, _):
            chunk = s + k * NS
            pltpu.sync_copy(srows, agg.at[pl.ds(chunk * ZROW, ZROW)])
            return 0
        lax.fori_loop(0, NZ, zc, 0)

        if with_rel:
            @pl.when(s == 0)
            def _():
                pltpu.sync_copy(rel_hbm, rel_sh)

        plsc.subcore_barrier()

        def mul():
            def mbody(i2, _):
                for v in range(2):
                    for jj in range(D // L):
                        r = 2 * i2 + v
                        sl = pl.ds(jj * L, L)
                        srows[r, sl] = srows[r, sl] * rrows[r, sl]
                return 0
            lax.fori_loop(0, EROW // 2, mbody, 0)

        def row(r, u, first=False):
            p = pidx[u]
            if not first:
                pltpu.make_async_copy(
                    pidx_hbm.at[r], p, sidx[u]).wait()

            @pl.when(r + 1 < base + RPT)
            def _():
                pltpu.async_copy(pidx_hbm.at[r + 1], pidx[1 - u], sidx[1 - u])
            cp0 = pltpu.async_copy(tab_hbm.at[p.at[0]], srows, se)
            if with_rel:
                cp1 = pltpu.async_copy(rel_sh.at[p.at[1]], rrows, sr)
            cp0.wait()
            if with_rel:
                cp1.wait()
                mul()
            pltpu.sync_copy(srows, agg.at[p.at[nf - 1]], add=True)

        pltpu.sync_copy(pidx_hbm.at[base], pi0)
        row(base, 0, first=True)

        def pair(k, _):
            r0 = base + 2 * k

            @pl.when(k > 0)
            def _():
                row(r0, 0)
            row(r0 + 1, 1)
            return 0
        lax.fori_loop(0, RPT // 2, pair, 0)

        plsc.subcore_barrier()

        def wb(k, _):
            chunk = s + k * NS
            sl = pl.ds(chunk * ZROW, ZROW)
            pltpu.sync_copy(agg.at[sl], out_hbm.at[c, sl])
            return 0
        lax.fori_loop(0, NZ, wb, 0)

    return pl.kernel(
        body,
        out_type=jax.ShapeDtypeStruct((NC, N_PAD, D), jnp.float32),
        mesh=_mesh,
        scratch_types=scratch,
    )


_sc_msg_pass = _make_sc_pass(with_rel=True)
_sc_agg_pass = _make_sc_pass(with_rel=False)


def _bn_relu(x, g, b):
    mu = jnp.mean(x, axis=0, keepdims=True)
    var = jnp.mean((x - mu) ** 2, axis=0, keepdims=True)
    return jnp.maximum((x - mu) / jnp.sqrt(var + 1e-5) * g + b, 0.0)


def _tc_proj_body(emb_h_ref, w_e_ref, b_e_ref, rel_wt_ref, emb_e_ref,
                  ent_out, rel_out):
    ent_out[...] = (
        jnp.dot(emb_h_ref[...], w_e_ref[...], preferred_element_type=jnp.float32)
        + b_e_ref[...]
    )
    rel_out[...] = jnp.dot(
        rel_wt_ref[...], emb_e_ref[...], preferred_element_type=jnp.float32
    )


def _tc_bn0_body(p_ref, ent_ref, relrow_ref, g_ref, b_ref, out_ref):
    agg = (p_ref[0, :N_ENT, :] + p_ref[1, :N_ENT, :]
           + ent_ref[...] * relrow_ref[...])
    out_ref[...] = _bn_relu(agg, g_ref[...], b_ref[...])


def _tc_head_body(p_ref, z_ref, rel_e_ref, w_rel_ref, subj_ref,
                  rel_ref, wtop_ref, wbot_ref, cb_ref, g1_ref, b1_ref,
                  gc_ref, bc_ref, h_out, q_out):
    z = z_ref[...]
    agg1 = p_ref[0, :N_ENT, :] + p_ref[1, :N_ENT, :] + z
    h1 = _bn_relu(agg1, g1_ref[...], b1_ref[...])
    hc = (
        jnp.dot(z, wtop_ref[...], preferred_element_type=jnp.float32)
        + jnp.dot(h1, wbot_ref[...], preferred_element_type=jnp.float32)
        + cb_ref[...]
    )
    h = _bn_relu(hc, gc_ref[...], bc_ref[...])
    h_out[...] = h

    rel2 = jnp.dot(rel_e_ref[...], w_rel_ref[...], preferred_element_type=jnp.float32)
    ohr = (rel_ref[...] == lax.broadcasted_iota(jnp.int32, (B, NUM_REL), 1))
    q_r = jnp.dot(ohr.astype(jnp.float32), rel2, preferred_element_type=jnp.float32)

    subj = subj_ref[...]
    acc = jnp.zeros((B, D), jnp.float32)
    blk = 2000
    for k in range(N_ENT // blk):
        iota = lax.broadcasted_iota(jnp.int32, (B, blk), 1) + k * blk
        oh = (subj == iota).astype(jnp.float32)
        acc = acc + jnp.dot(oh, h[k * blk:(k + 1) * blk, :],
                            preferred_element_type=jnp.float32)
    q_out[...] = acc * q_r


def _tc_score_body(q_ref, h_ref, out_ref):
    out_ref[...] = lax.dot_general(
        q_ref[...], h_ref[...],
        (((1,), (1,)), ((), ())),
        preferred_element_type=jnp.float32,
    )


def kernel(edge_index, edge_type, subj, rel, emb_h, emb_e, W_e, b_e, rel_wt,
           w_rel, bn0_g, bn0_b, bn1_g, bn1_b, concat_W, concat_b, bnc_g, bnc_b):
    i32 = jnp.int32
    src_f = jnp.concatenate([edge_index[0].astype(i32), jnp.zeros((PAD_E,), i32)])
    dst_f = jnp.concatenate([edge_index[1].astype(i32), jnp.full((PAD_E,), N_ENT, i32)])
    et_f = jnp.concatenate([edge_type.astype(i32), jnp.zeros((PAD_E,), i32)])

    def _pack(arrs):
        parts = [a.reshape(NW * RPT, 1, EROW) for a in arrs]
        return jnp.concatenate(parts, axis=1)

    pidx1 = _pack([src_f, et_f, dst_f])
    pidx2 = _pack([src_f, dst_f])
    subj2d = subj.astype(i32).reshape(B, 1)
    rel2d = rel.astype(i32).reshape(B, 1)

    ent, rel_embed = pl.pallas_call(
        _tc_proj_body,
        out_shape=(
            jax.ShapeDtypeStruct((N_ENT, D), jnp.float32),
            jax.ShapeDtypeStruct((NUM_REL, D), jnp.float32),
        ),
    )(emb_h, W_e, b_e.reshape(1, D), rel_wt, emb_e)

    p1 = _sc_msg_pass(ent, rel_embed, pidx1)

    zero_out = pl.pallas_call(
        _tc_bn0_body,
        out_shape=jax.ShapeDtypeStruct((N_ENT, D), jnp.float32),
    )(p1, ent, rel_embed[NUM_REL - 1:NUM_REL], bn0_g.reshape(1, D),
      bn0_b.reshape(1, D))

    p2 = _sc_agg_pass(zero_out, pidx2)

    h, q = pl.pallas_call(
        _tc_head_body,
        out_shape=(
            jax.ShapeDtypeStruct((N_ENT, D), jnp.float32),
            jax.ShapeDtypeStruct((B, D), jnp.float32),
        ),
    )(p2, zero_out, rel_embed, w_rel, subj2d, rel2d,
      concat_W[:D], concat_W[D:], concat_b.reshape(1, D),
      bn1_g.reshape(1, D), bn1_b.reshape(1, D),
      bnc_g.reshape(1, D), bnc_b.reshape(1, D))

    score = pl.pallas_call(
        _tc_score_body,
        out_shape=jax.ShapeDtypeStruct((B, N_ENT), jnp.float32),
    )(q, h)
    return score
